# baseline (speedup 1.0000x reference)
import functools
import math

import jax
import jax.numpy as jnp
from jax import lax
from jax.experimental import pallas as pl
from jax.experimental.pallas import tpu as pltpu
from jax.experimental.pallas import tpu_sc as plsc

N_ATOMS = 10000
N_EDGES = 160000
N_FEAT = 128
N_RADIAL = 20
S = 4
ROW = S * N_FEAT
CNEG = -1.0 / math.sqrt(3.0)

NC = 2
NS = 16
NW = NC * NS
E_PC = N_EDGES // NW
G = 40
NBLK = E_PC // G
PAD = 1000
E_PAD = N_EDGES + PAD
KMAX = E_PAD // G
TCB = 1000
DUMMY = N_ATOMS


def _wfull_body(r_ref, cut_ref, w_ref, b_ref, o_ref):
    r = r_ref[...]
    acc = jnp.dot(r, w_ref[...], preferred_element_type=jnp.float32)
    o_ref[...] = (acc + b_ref[...]) * cut_ref[...]


def _wfull(radial_p, cutoff_p, W, b):
    grid = (E_PAD // TCB,)
    return pl.pallas_call(
        _wfull_body,
        grid=grid,
        in_specs=[
            pl.BlockSpec((TCB, N_RADIAL), lambda i: (i, 0)),
            pl.BlockSpec((TCB, 1), lambda i: (i, 0)),
            pl.BlockSpec((N_RADIAL, 2 * N_FEAT), lambda i: (0, 0)),
            pl.BlockSpec((1, 2 * N_FEAT), lambda i: (0, 0)),
        ],
        out_specs=pl.BlockSpec((TCB, 2 * N_FEAT), lambda i: (i, 0)),
        out_shape=jax.ShapeDtypeStruct((E_PAD, 2 * N_FEAT), jnp.float32),
    )(radial_p, cutoff_p, W, b)


def _splat16(val_i32):
    return jnp.full((16,), val_i32, jnp.int32)


def _sc_body(x_hbm, wf_hbm, dir_hbm, idxi_hbm, idxj_hbm, ya_hbm, yb_hbm,
             idxi_v, idxj_v, xj_v0, xj_v1, w_v0, w_v1, dir_v0, dir_v1,
             acc_v, stage_v, zr_v, mprev_v, idxi2_v, idxj2_v,
             sem_g, sem_l, sem_st, sem_f):
    cid = lax.axis_index("c")
    wid = cid * NS + lax.axis_index("s")
    base = wid * E_PC
    end_base = base + E_PC

    @pl.when(cid == 0)
    def _():
        _sc_half(x_hbm, wf_hbm, dir_hbm, idxi_hbm, idxj_hbm, ya_hbm,
                 idxi_v, idxj_v, xj_v0, xj_v1, w_v0, w_v1, dir_v0, dir_v1,
                 acc_v, stage_v, zr_v, mprev_v, idxi2_v, idxj2_v,
                 sem_g, sem_l, sem_st, sem_f, wid, base, end_base)

    @pl.when(cid == 1)
    def _():
        _sc_half(x_hbm, wf_hbm, dir_hbm, idxi_hbm, idxj_hbm, yb_hbm,
                 idxi_v, idxj_v, xj_v0, xj_v1, w_v0, w_v1, dir_v0, dir_v1,
                 acc_v, stage_v, zr_v, mprev_v, idxi2_v, idxj2_v,
                 sem_g, sem_l, sem_st, sem_f, wid, base, end_base)


def _sc_half(x_hbm, wf_hbm, dir_hbm, idxi_hbm, idxj_hbm, y_hbm,
             idxi_v, idxj_v, xj_v0, xj_v1, w_v0, w_v1, dir_v0, dir_v1,
             acc_v, stage_v, zr_v, mprev_v, idxi2_v, idxj2_v,
             sem_g, sem_l, sem_st, sem_f, wid, base, end_base):

    def _z8(i, _):
        zr_v[pl.ds(i * 16, 16)] = jnp.zeros((16,), jnp.float32)
        return 0
    lax.fori_loop(0, (8 * ROW) // 16, _z8, 0)

    def _za(i, _):
        acc_v[pl.ds(i * 16, 16)] = jnp.zeros((16,), jnp.float32)
        return 0
    lax.fori_loop(0, ROW // 16, _za, 0)

    @pl.when(wid > 0)
    def _():
        pltpu.async_copy(idxi_hbm.at[pl.ds(base - 16, 16)], mprev_v,
                         sem_l).wait()
    m = jnp.where(wid > 0, mprev_v[pl.ds(0, 16)][15], jnp.int32(-1))

    pltpu.async_copy(stage_v.at[0], y_hbm.at[pl.ds(DUMMY * ROW, ROW)],
                     sem_st.at[0])
    pltpu.async_copy(stage_v.at[1], y_hbm.at[pl.ds(DUMMY * ROW, ROW)],
                     sem_st.at[1])

    pltpu.async_copy(idxi_hbm.at[pl.ds(base, E_PC)], idxi_v, sem_l)
    pltpu.async_copy(idxj_hbm.at[pl.ds(base, E_PC)], idxj_v, sem_l)
    pltpu.make_async_copy(idxi_hbm.at[pl.ds(base, E_PC)], idxi_v,
                          sem_l).wait()
    pltpu.make_async_copy(idxj_hbm.at[pl.ds(base, E_PC)], idxj_v,
                          sem_l).wait()

    def zero_fill(wstart, cur):
        nfill = cur - wstart
        n8 = nfill // 8

        def _gap8(t, _):
            pltpu.async_copy(zr_v.at[pl.ds(0, 8 * ROW)],
                             y_hbm.at[pl.ds((wstart + t * 8) * ROW, 8 * ROW)],
                             sem_l).wait()
            return 0
        lax.fori_loop(0, n8, _gap8, 0)

        def _gap1(t, _):
            pltpu.async_copy(
                zr_v.at[pl.ds(0, ROW)],
                y_hbm.at[pl.ds((wstart + n8 * 8 + t) * ROW, ROW)],
                sem_l).wait()
            return 0
        lax.fori_loop(0, nfill - n8 * 8, _gap1, 0)

    def flush(cur, wstart, par):
        valid = cur != m
        zero_fill(wstart, cur)

        tgt = jnp.where(valid, cur, jnp.int32(DUMMY))

        def _emit(slot):
            pltpu.make_async_copy(stage_v.at[slot],
                                  y_hbm.at[pl.ds(DUMMY * ROW, ROW)],
                                  sem_st.at[slot]).wait()

            def _cp(i, _):
                v = acc_v[pl.ds(i * 16, 16)]
                stage_v[slot, pl.ds(i * 16, 16)] = v
                acc_v[pl.ds(i * 16, 16)] = jnp.zeros((16,), jnp.float32)
                return 0
            lax.fori_loop(0, ROW // 16, _cp, 0)
            pltpu.async_copy(stage_v.at[slot],
                             y_hbm.at[pl.ds(tgt * ROW, ROW)], sem_st.at[slot])

        @pl.when(par == 0)
        def _():
            _emit(0)

        @pl.when(par == 1)
        def _():
            _emit(1)

        wstart2 = jnp.where(valid, cur + 1, wstart)
        return wstart2, 1 - par

    def accumulate(p, xj_v, w_v, dir_v):
        d0 = plsc.load_gather(dir_v, [_splat16(p * S + 0)])
        d1 = plsc.load_gather(dir_v, [_splat16(p * S + 1)])
        d2 = plsc.load_gather(dir_v, [_splat16(p * S + 2)])
        d3 = plsc.load_gather(dir_v, [_splat16(p * S + 3)])

        cd1 = CNEG * d1
        cd2 = CNEG * d2
        cd3 = CNEG * d3
        for j in range(N_FEAT // 16):
            o = j * 16
            xj0 = xj_v[p, pl.ds(o, 16)]
            xj1 = xj_v[p, pl.ds(N_FEAT + o, 16)]
            xj2 = xj_v[p, pl.ds(2 * N_FEAT + o, 16)]
            xj3 = xj_v[p, pl.ds(3 * N_FEAT + o, 16)]
            w0 = w_v[p, pl.ds(o, 16)]
            w1 = w_v[p, pl.ds(N_FEAT + o, 16)]
            u = d0 * w0
            s_ = w1 * xj0
            c0 = u * xj0 + w1 * (cd1 * xj1 + cd2 * xj2 + cd3 * xj3)
            c1 = u * xj1 + d1 * s_
            c2 = u * xj2 + d2 * s_
            c3 = u * xj3 + d3 * s_
            plsc.addupdate(acc_v.at[pl.ds(o, 16)], c0)
            plsc.addupdate(acc_v.at[pl.ds(N_FEAT + o, 16)], c1)
            plsc.addupdate(acc_v.at[pl.ds(2 * N_FEAT + o, 16)], c2)
            plsc.addupdate(acc_v.at[pl.ds(3 * N_FEAT + o, 16)], c3)

    def fetch_block(koff):
        pltpu.async_copy(wf_hbm.at[pl.ds(koff, G)], w_v0, sem_l)
        pltpu.async_copy(dir_hbm.at[pl.ds(koff * S, G * S)], dir_v0, sem_l)
        pltpu.async_copy(x_hbm.at[idxj2_v], xj_v0, sem_g)
        pltpu.make_async_copy(wf_hbm.at[pl.ds(koff, G)], w_v0, sem_l).wait()
        pltpu.make_async_copy(dir_hbm.at[pl.ds(koff * S, G * S)],
                              dir_v0, sem_l).wait()
        pltpu.make_async_copy(x_hbm.at[idxj2_v], xj_v0, sem_g).wait()

    def issue(k, w_r, dir_r, xj_r, sem):
        koff = base + k * G
        pltpu.async_copy(wf_hbm.at[pl.ds(koff, G)], w_r, sem)
        pltpu.async_copy(dir_hbm.at[pl.ds(koff * S, G * S)], dir_r, sem)
        pltpu.async_copy(x_hbm.at[idxj_v.at[pl.ds(k * G, G)]], xj_r, sem)

    def drain(k, w_r, dir_r, xj_r, sem):
        koff = base + k * G
        pltpu.make_async_copy(wf_hbm.at[pl.ds(koff, G)], w_r, sem).wait()
        pltpu.make_async_copy(dir_hbm.at[pl.ds(koff * S, G * S)],
                              dir_r, sem).wait()
        pltpu.make_async_copy(x_hbm.at[idxj_v.at[pl.ds(k * G, G)]],
                              xj_r, sem).wait()

    def process_block(k, st, xj_r, w_r, dir_r):
        def p1_edge(p, st):
            cur, wstart, par = st
            n = plsc.load_gather(idxi_v, [_splat16(k * G + p)])[0]
            boundary = n != cur

            def _fl(args):
                c_, w_, pa_ = args
                w2, pa2 = flush(c_, w_, pa_)
                return n, w2, pa2
            cur, wstart, par = lax.cond(
                boundary, _fl, lambda a: a, (cur, wstart, par))
            accumulate(p, xj_r, w_r, dir_r)
            return cur, wstart, par

        return lax.fori_loop(0, G, p1_edge, st)

    cur0 = m
    wstart0 = jnp.where(wid > 0, m + 1, jnp.int32(0))
    st = (cur0, wstart0, jnp.int32(0))

    issue(0, w_v0, dir_v0, xj_v0, sem_f.at[0])

    def pair(i, st):
        k0 = i * 2
        issue(k0 + 1, w_v1, dir_v1, xj_v1, sem_f.at[1])
        drain(k0, w_v0, dir_v0, xj_v0, sem_f.at[0])
        st = process_block(k0, st, xj_v0, w_v0, dir_v0)
        issue(k0 + 2, w_v0, dir_v0, xj_v0, sem_f.at[0])
        drain(k0 + 1, w_v1, dir_v1, xj_v1, sem_f.at[1])
        st = process_block(k0 + 1, st, xj_v1, w_v1, dir_v1)
        return st

    st = lax.fori_loop(0, (NBLK - 1) // 2, pair, st)
    drain(NBLK - 1, w_v0, dir_v0, xj_v0, sem_f.at[0])
    cur, wstart, par = process_block(NBLK - 1, st, xj_v0, w_v0, dir_v0)

    def p2_edge(p, st):
        done, cur, wstart, par = st
        n = plsc.load_gather(idxi2_v, [_splat16(p)])[0]
        active = done == 0
        boundary = (n != cur) & active

        def _fl(args):
            c_, w_, pa_ = args
            w2, pa2 = flush(c_, w_, pa_)
            return c_, w2, pa2
        cur, wstart, par = lax.cond(
            boundary, _fl, lambda a: a, (cur, wstart, par))
        done = jnp.where(active & (boundary | (cur == m)),
                         jnp.int32(1), done)

        @pl.when(done == 0)
        def _():
            accumulate(p, xj_v0, w_v0, dir_v0)
        return done, cur, wstart, par

    def p2_block(k, st):
        done, cur, wstart, par = st

        def _act(args):
            done, cur, wstart, par = args
            koff = k * G
            pltpu.async_copy(idxi_hbm.at[pl.ds(koff, G)], idxi2_v,
                             sem_l).wait()
            pltpu.async_copy(idxj_hbm.at[pl.ds(koff, G)], idxj2_v,
                             sem_l).wait()
            fetch_block(koff)
            return lax.fori_loop(0, G, p2_edge, (done, cur, wstart, par))

        return lax.cond(done == 0, _act, lambda a: a,
                        (done, cur, wstart, par))

    done, cur, wstart, par = lax.fori_loop(
        end_base // G, KMAX, p2_block,
        (jnp.int32(0), cur, wstart, par))

    @pl.when(wid == NW - 1)
    def _():
        zero_fill(wstart, jnp.int32(N_ATOMS))

    pltpu.make_async_copy(stage_v.at[0],
                          y_hbm.at[pl.ds(DUMMY * ROW, ROW)], sem_st.at[0]).wait()
    pltpu.make_async_copy(stage_v.at[1],
                          y_hbm.at[pl.ds(DUMMY * ROW, ROW)], sem_st.at[1]).wait()


def _sc_conv(x2, wfull, dir_p, idxi_p, idxj_p):
    mesh = plsc.VectorSubcoreMesh(core_axis_name="c", subcore_axis_name="s",
                                  num_cores=NC, num_subcores=NS)
    fn = pl.kernel(
        _sc_body,
        out_type=(jax.ShapeDtypeStruct(((N_ATOMS + 8) * ROW,), jnp.float32),
                  jax.ShapeDtypeStruct(((N_ATOMS + 8) * ROW,), jnp.float32)),
        mesh=mesh,
        compiler_params=pltpu.CompilerParams(needs_layout_passes=False),
        scratch_types=[
            pltpu.VMEM((E_PC,), jnp.int32),
            pltpu.VMEM((E_PC,), jnp.int32),
            pltpu.VMEM((G, ROW), jnp.float32),
            pltpu.VMEM((G, ROW), jnp.float32),
            pltpu.VMEM((G, 2 * N_FEAT), jnp.float32),
            pltpu.VMEM((G, 2 * N_FEAT), jnp.float32),
            pltpu.VMEM((G * S,), jnp.float32),
            pltpu.VMEM((G * S,), jnp.float32),
            pltpu.VMEM((ROW,), jnp.float32),
            pltpu.VMEM((2, ROW), jnp.float32),
            pltpu.VMEM((8 * ROW,), jnp.float32),
            pltpu.VMEM((16,), jnp.int32),
            pltpu.VMEM((G,), jnp.int32),
            pltpu.VMEM((G,), jnp.int32),
            pltpu.SemaphoreType.DMA,
            pltpu.SemaphoreType.DMA,
            pltpu.SemaphoreType.DMA((2,)),
            pltpu.SemaphoreType.DMA((2,)),
        ],
    )
    return fn(x2, wfull, dir_p, idxi_p, idxj_p)


def kernel(x, radial_ij, dir_ij, cutoff_ij, idx_i, idx_j, W, b):
    x2 = x.reshape(N_ATOMS, ROW)
    pad = PAD
    radial_p = jnp.concatenate(
        [radial_ij, jnp.zeros((pad, N_RADIAL), jnp.float32)])
    cutoff_p = jnp.concatenate([cutoff_ij, jnp.zeros((pad, 1), jnp.float32)])
    dir_p = jnp.concatenate([dir_ij, jnp.zeros((pad, S), jnp.float32)]).reshape(-1)
    idxi_p = jnp.concatenate(
        [idx_i.astype(jnp.int32),
         jnp.full((pad,), N_ATOMS, jnp.int32)])
    idxj_p = jnp.concatenate(
        [idx_j.astype(jnp.int32), jnp.zeros((pad,), jnp.int32)])
    b2 = b.reshape(1, 2 * N_FEAT)

    wfull = _wfull(radial_p, cutoff_p, W, b2)
    ya, yb = _sc_conv(x2, wfull, dir_p, idxi_p, idxj_p)
    split = idx_i[N_EDGES // 2 - 1].astype(jnp.int32) + 1
    ya2 = ya[:N_ATOMS * ROW].reshape(N_ATOMS, S, N_FEAT)
    yb2 = yb[:N_ATOMS * ROW].reshape(N_ATOMS, S, N_FEAT)
    rows = jnp.arange(N_ATOMS, dtype=jnp.int32) < split
    return jnp.where(rows[:, None, None], ya2, yb2)


# speedup vs baseline: 1.0727x; 1.0727x over previous
import functools
import math

import jax
import jax.numpy as jnp
from jax import lax
from jax.experimental import pallas as pl
from jax.experimental.pallas import tpu as pltpu
from jax.experimental.pallas import tpu_sc as plsc

N_ATOMS = 10000
N_EDGES = 160000
N_FEAT = 128
N_RADIAL = 20
S = 4
ROW = S * N_FEAT
CNEG = -1.0 / math.sqrt(3.0)

NC = 2
NS = 16
NW = NC * NS
E_PC = N_EDGES // NW
G = 40
NBLK = E_PC // G
PAD = 1000
E_PAD = N_EDGES + PAD
KMAX = E_PAD // G
TCB = 7000
DUMMY = N_ATOMS


def _wfull_body(r_ref, cut_ref, w_ref, b_ref, o_ref):
    r = r_ref[...]
    acc = jnp.dot(r, w_ref[...], preferred_element_type=jnp.float32)
    o_ref[...] = (acc + b_ref[...]) * cut_ref[...]


def _wfull(radial_p, cutoff_p, W, b):
    grid = (E_PAD // TCB,)
    return pl.pallas_call(
        _wfull_body,
        grid=grid,
        in_specs=[
            pl.BlockSpec((TCB, N_RADIAL), lambda i: (i, 0)),
            pl.BlockSpec((TCB, 1), lambda i: (i, 0)),
            pl.BlockSpec((N_RADIAL, 2 * N_FEAT), lambda i: (0, 0)),
            pl.BlockSpec((1, 2 * N_FEAT), lambda i: (0, 0)),
        ],
        out_specs=pl.BlockSpec((TCB, 2 * N_FEAT), lambda i: (i, 0)),
        out_shape=jax.ShapeDtypeStruct((E_PAD, 2 * N_FEAT), jnp.float32),
    )(radial_p, cutoff_p, W, b)


def _splat16(val_i32):
    return jnp.full((16,), val_i32, jnp.int32)


def _sc_body(x_hbm, wf_hbm, dir_hbm, idxi_hbm, idxj_hbm, ya_hbm, yb_hbm,
             idxi_v, idxj_v, xj_v0, xj_v1, w_v0, w_v1, dir_v0, dir_v1,
             acc_v, stage_v, zr_v, mprev_v, idxi2_v, idxj2_v,
             sem_g, sem_l, sem_st, sem_f):
    cid = lax.axis_index("c")
    wid = cid * NS + lax.axis_index("s")
    base = wid * E_PC
    end_base = base + E_PC

    @pl.when(cid == 0)
    def _():
        _sc_half(x_hbm, wf_hbm, dir_hbm, idxi_hbm, idxj_hbm, ya_hbm,
                 idxi_v, idxj_v, xj_v0, xj_v1, w_v0, w_v1, dir_v0, dir_v1,
                 acc_v, stage_v, zr_v, mprev_v, idxi2_v, idxj2_v,
                 sem_g, sem_l, sem_st, sem_f, wid, base, end_base)

    @pl.when(cid == 1)
    def _():
        _sc_half(x_hbm, wf_hbm, dir_hbm, idxi_hbm, idxj_hbm, yb_hbm,
                 idxi_v, idxj_v, xj_v0, xj_v1, w_v0, w_v1, dir_v0, dir_v1,
                 acc_v, stage_v, zr_v, mprev_v, idxi2_v, idxj2_v,
                 sem_g, sem_l, sem_st, sem_f, wid, base, end_base)


def _sc_half(x_hbm, wf_hbm, dir_hbm, idxi_hbm, idxj_hbm, y_hbm,
             idxi_v, idxj_v, xj_v0, xj_v1, w_v0, w_v1, dir_v0, dir_v1,
             acc_v, stage_v, zr_v, mprev_v, idxi2_v, idxj2_v,
             sem_g, sem_l, sem_st, sem_f, wid, base, end_base):

    def _z8(i, _):
        zr_v[pl.ds(i * 16, 16)] = jnp.zeros((16,), jnp.float32)
        return 0
    lax.fori_loop(0, (8 * ROW) // 16, _z8, 0)

    def _za(i, _):
        acc_v[pl.ds(i * 16, 16)] = jnp.zeros((16,), jnp.float32)
        return 0
    lax.fori_loop(0, ROW // 16, _za, 0)

    @pl.when(wid > 0)
    def _():
        pltpu.async_copy(idxi_hbm.at[pl.ds(base - 16, 16)], mprev_v,
                         sem_l).wait()
    m = jnp.where(wid > 0, mprev_v[pl.ds(0, 16)][15], jnp.int32(-1))

    pltpu.async_copy(stage_v.at[0], y_hbm.at[pl.ds(DUMMY * ROW, ROW)],
                     sem_st.at[0])
    pltpu.async_copy(stage_v.at[1], y_hbm.at[pl.ds(DUMMY * ROW, ROW)],
                     sem_st.at[1])

    pltpu.async_copy(idxi_hbm.at[pl.ds(base, E_PC)], idxi_v, sem_l)
    pltpu.async_copy(idxj_hbm.at[pl.ds(base, E_PC)], idxj_v, sem_l)
    pltpu.make_async_copy(idxi_hbm.at[pl.ds(base, E_PC)], idxi_v,
                          sem_l).wait()
    pltpu.make_async_copy(idxj_hbm.at[pl.ds(base, E_PC)], idxj_v,
                          sem_l).wait()

    def zero_fill(wstart, cur):
        nfill = cur - wstart
        n8 = nfill // 8

        def _gap8(t, _):
            pltpu.async_copy(zr_v.at[pl.ds(0, 8 * ROW)],
                             y_hbm.at[pl.ds((wstart + t * 8) * ROW, 8 * ROW)],
                             sem_l).wait()
            return 0
        lax.fori_loop(0, n8, _gap8, 0)

        def _gap1(t, _):
            pltpu.async_copy(
                zr_v.at[pl.ds(0, ROW)],
                y_hbm.at[pl.ds((wstart + n8 * 8 + t) * ROW, ROW)],
                sem_l).wait()
            return 0
        lax.fori_loop(0, nfill - n8 * 8, _gap1, 0)

    def flush(cur, wstart, par):
        valid = cur != m
        zero_fill(wstart, cur)

        tgt = jnp.where(valid, cur, jnp.int32(DUMMY))

        def _emit(slot):
            pltpu.make_async_copy(stage_v.at[slot],
                                  y_hbm.at[pl.ds(DUMMY * ROW, ROW)],
                                  sem_st.at[slot]).wait()

            def _cp(i, _):
                v = acc_v[pl.ds(i * 16, 16)]
                stage_v[slot, pl.ds(i * 16, 16)] = v
                acc_v[pl.ds(i * 16, 16)] = jnp.zeros((16,), jnp.float32)
                return 0
            lax.fori_loop(0, ROW // 16, _cp, 0)
            pltpu.async_copy(stage_v.at[slot],
                             y_hbm.at[pl.ds(tgt * ROW, ROW)], sem_st.at[slot])

        @pl.when(par == 0)
        def _():
            _emit(0)

        @pl.when(par == 1)
        def _():
            _emit(1)

        wstart2 = jnp.where(valid, cur + 1, wstart)
        return wstart2, 1 - par

    def accumulate(p, xj_v, w_v, dir_v):
        d0 = plsc.load_gather(dir_v, [_splat16(p * S + 0)])
        d1 = plsc.load_gather(dir_v, [_splat16(p * S + 1)])
        d2 = plsc.load_gather(dir_v, [_splat16(p * S + 2)])
        d3 = plsc.load_gather(dir_v, [_splat16(p * S + 3)])

        cd1 = CNEG * d1
        cd2 = CNEG * d2
        cd3 = CNEG * d3
        for j in range(N_FEAT // 16):
            o = j * 16
            xj0 = xj_v[p, 0, pl.ds(o, 16)]
            xj1 = xj_v[p, 1, pl.ds(o, 16)]
            xj2 = xj_v[p, 2, pl.ds(o, 16)]
            xj3 = xj_v[p, 3, pl.ds(o, 16)]
            w0 = w_v[p, pl.ds(o, 16)]
            w1 = w_v[p, pl.ds(N_FEAT + o, 16)]
            u = d0 * w0
            s_ = w1 * xj0
            c0 = u * xj0 + w1 * (cd1 * xj1 + cd2 * xj2 + cd3 * xj3)
            c1 = u * xj1 + d1 * s_
            c2 = u * xj2 + d2 * s_
            c3 = u * xj3 + d3 * s_
            plsc.addupdate(acc_v.at[pl.ds(o, 16)], c0)
            plsc.addupdate(acc_v.at[pl.ds(N_FEAT + o, 16)], c1)
            plsc.addupdate(acc_v.at[pl.ds(2 * N_FEAT + o, 16)], c2)
            plsc.addupdate(acc_v.at[pl.ds(3 * N_FEAT + o, 16)], c3)

    def fetch_block(koff):
        pltpu.async_copy(wf_hbm.at[pl.ds(koff, G)], w_v0, sem_l)
        pltpu.async_copy(dir_hbm.at[pl.ds(koff * S, G * S)], dir_v0, sem_l)
        pltpu.async_copy(x_hbm.at[idxj2_v], xj_v0, sem_g)
        pltpu.make_async_copy(wf_hbm.at[pl.ds(koff, G)], w_v0, sem_l).wait()
        pltpu.make_async_copy(dir_hbm.at[pl.ds(koff * S, G * S)],
                              dir_v0, sem_l).wait()
        pltpu.make_async_copy(x_hbm.at[idxj2_v], xj_v0, sem_g).wait()

    def issue(k, w_r, dir_r, xj_r, sem):
        koff = base + k * G
        pltpu.async_copy(wf_hbm.at[pl.ds(koff, G)], w_r, sem)
        pltpu.async_copy(dir_hbm.at[pl.ds(koff * S, G * S)], dir_r, sem)
        pltpu.async_copy(x_hbm.at[idxj_v.at[pl.ds(k * G, G)]], xj_r, sem)

    def drain(k, w_r, dir_r, xj_r, sem):
        koff = base + k * G
        pltpu.make_async_copy(wf_hbm.at[pl.ds(koff, G)], w_r, sem).wait()
        pltpu.make_async_copy(dir_hbm.at[pl.ds(koff * S, G * S)],
                              dir_r, sem).wait()
        pltpu.make_async_copy(x_hbm.at[idxj_v.at[pl.ds(k * G, G)]],
                              xj_r, sem).wait()

    def process_block(k, st, xj_r, w_r, dir_r):
        def p1_edge(p, st):
            cur, wstart, par = st
            n = plsc.load_gather(idxi_v, [_splat16(k * G + p)])[0]
            boundary = n != cur

            def _fl(args):
                c_, w_, pa_ = args
                w2, pa2 = flush(c_, w_, pa_)
                return n, w2, pa2
            cur, wstart, par = lax.cond(
                boundary, _fl, lambda a: a, (cur, wstart, par))
            accumulate(p, xj_r, w_r, dir_r)
            return cur, wstart, par

        return lax.fori_loop(0, G, p1_edge, st)

    cur0 = m
    wstart0 = jnp.where(wid > 0, m + 1, jnp.int32(0))
    st = (cur0, wstart0, jnp.int32(0))

    issue(0, w_v0, dir_v0, xj_v0, sem_f.at[0])

    def pair(i, st):
        k0 = i * 2
        issue(k0 + 1, w_v1, dir_v1, xj_v1, sem_f.at[1])
        drain(k0, w_v0, dir_v0, xj_v0, sem_f.at[0])
        st = process_block(k0, st, xj_v0, w_v0, dir_v0)
        issue(k0 + 2, w_v0, dir_v0, xj_v0, sem_f.at[0])
        drain(k0 + 1, w_v1, dir_v1, xj_v1, sem_f.at[1])
        st = process_block(k0 + 1, st, xj_v1, w_v1, dir_v1)
        return st

    st = lax.fori_loop(0, (NBLK - 1) // 2, pair, st)
    drain(NBLK - 1, w_v0, dir_v0, xj_v0, sem_f.at[0])
    cur, wstart, par = process_block(NBLK - 1, st, xj_v0, w_v0, dir_v0)

    def p2_edge(p, st):
        done, cur, wstart, par = st
        n = plsc.load_gather(idxi2_v, [_splat16(p)])[0]
        active = done == 0
        boundary = (n != cur) & active

        def _fl(args):
            c_, w_, pa_ = args
            w2, pa2 = flush(c_, w_, pa_)
            return c_, w2, pa2
        cur, wstart, par = lax.cond(
            boundary, _fl, lambda a: a, (cur, wstart, par))
        done = jnp.where(active & (boundary | (cur == m)),
                         jnp.int32(1), done)

        @pl.when(done == 0)
        def _():
            accumulate(p, xj_v0, w_v0, dir_v0)
        return done, cur, wstart, par

    def p2_block(k, st):
        done, cur, wstart, par = st

        def _act(args):
            done, cur, wstart, par = args
            koff = k * G
            pltpu.async_copy(idxi_hbm.at[pl.ds(koff, G)], idxi2_v,
                             sem_l).wait()
            pltpu.async_copy(idxj_hbm.at[pl.ds(koff, G)], idxj2_v,
                             sem_l).wait()
            fetch_block(koff)
            return lax.fori_loop(0, G, p2_edge, (done, cur, wstart, par))

        return lax.cond(done == 0, _act, lambda a: a,
                        (done, cur, wstart, par))

    done, cur, wstart, par = lax.fori_loop(
        end_base // G, KMAX, p2_block,
        (jnp.int32(0), cur, wstart, par))

    @pl.when(wid == NW - 1)
    def _():
        zero_fill(wstart, jnp.int32(N_ATOMS))

    pltpu.make_async_copy(stage_v.at[0],
                          y_hbm.at[pl.ds(DUMMY * ROW, ROW)], sem_st.at[0]).wait()
    pltpu.make_async_copy(stage_v.at[1],
                          y_hbm.at[pl.ds(DUMMY * ROW, ROW)], sem_st.at[1]).wait()


def _sc_conv(x2, wfull, dir_p, idxi_p, idxj_p):
    mesh = plsc.VectorSubcoreMesh(core_axis_name="c", subcore_axis_name="s",
                                  num_cores=NC, num_subcores=NS)
    fn = pl.kernel(
        _sc_body,
        out_type=(jax.ShapeDtypeStruct(((N_ATOMS + 8) * ROW,), jnp.float32),
                  jax.ShapeDtypeStruct(((N_ATOMS + 8) * ROW,), jnp.float32)),
        mesh=mesh,
        compiler_params=pltpu.CompilerParams(needs_layout_passes=False),
        scratch_types=[
            pltpu.VMEM((E_PC,), jnp.int32),
            pltpu.VMEM((E_PC,), jnp.int32),
            pltpu.VMEM((G, S, N_FEAT), jnp.float32),
            pltpu.VMEM((G, S, N_FEAT), jnp.float32),
            pltpu.VMEM((G, 2 * N_FEAT), jnp.float32),
            pltpu.VMEM((G, 2 * N_FEAT), jnp.float32),
            pltpu.VMEM((G * S,), jnp.float32),
            pltpu.VMEM((G * S,), jnp.float32),
            pltpu.VMEM((ROW,), jnp.float32),
            pltpu.VMEM((2, ROW), jnp.float32),
            pltpu.VMEM((8 * ROW,), jnp.float32),
            pltpu.VMEM((16,), jnp.int32),
            pltpu.VMEM((G,), jnp.int32),
            pltpu.VMEM((G,), jnp.int32),
            pltpu.SemaphoreType.DMA,
            pltpu.SemaphoreType.DMA,
            pltpu.SemaphoreType.DMA((2,)),
            pltpu.SemaphoreType.DMA((2,)),
        ],
    )
    return fn(x2, wfull, dir_p, idxi_p, idxj_p)


def kernel(x, radial_ij, dir_ij, cutoff_ij, idx_i, idx_j, W, b):
    pad = PAD
    radial_p = jnp.concatenate(
        [radial_ij, jnp.zeros((pad, N_RADIAL), jnp.float32)])
    cutoff_p = jnp.concatenate([cutoff_ij, jnp.zeros((pad, 1), jnp.float32)])
    dir_p = jnp.concatenate([dir_ij, jnp.zeros((pad, S), jnp.float32)]).reshape(-1)
    idxi_p = jnp.concatenate(
        [idx_i.astype(jnp.int32),
         jnp.full((pad,), N_ATOMS, jnp.int32)])
    idxj_p = jnp.concatenate(
        [idx_j.astype(jnp.int32), jnp.zeros((pad,), jnp.int32)])
    b2 = b.reshape(1, 2 * N_FEAT)

    wfull = _wfull(radial_p, cutoff_p, W, b2)
    ya, yb = _sc_conv(x, wfull, dir_p, idxi_p, idxj_p)
    split = idx_i[N_EDGES // 2 - 1].astype(jnp.int32) + 1
    ya2 = ya[:N_ATOMS * ROW].reshape(N_ATOMS, S, N_FEAT)
    yb2 = yb[:N_ATOMS * ROW].reshape(N_ATOMS, S, N_FEAT)
    rows = jnp.arange(N_ATOMS, dtype=jnp.int32) < split
    return jnp.where(rows[:, None, None], ya2, yb2)


# speedup vs baseline: 1.0728x; 1.0001x over previous
import functools
import math

import jax
import jax.numpy as jnp
from jax import lax
from jax.experimental import pallas as pl
from jax.experimental.pallas import tpu as pltpu
from jax.experimental.pallas import tpu_sc as plsc

N_ATOMS = 10000
N_EDGES = 160000
N_FEAT = 128
N_RADIAL = 20
S = 4
ROW = S * N_FEAT
CNEG = -1.0 / math.sqrt(3.0)

NC = 2
NS = 16
NW = NC * NS
E_PC = N_EDGES // NW
G = 40
NBLK = E_PC // G
PAD = 1000
E_PAD = N_EDGES + PAD
KMAX = E_PAD // G
TCB = 7000
DUMMY = N_ATOMS


def _wfull_body(r_ref, cut_ref, w_ref, b_ref, o_ref):
    r = r_ref[...]
    acc = jnp.dot(r, w_ref[...], preferred_element_type=jnp.float32)
    o_ref[...] = (acc + b_ref[...]) * cut_ref[...]


def _wfull(radial_p, cutoff_p, W, b):
    grid = (E_PAD // TCB,)
    return pl.pallas_call(
        _wfull_body,
        grid=grid,
        in_specs=[
            pl.BlockSpec((TCB, N_RADIAL), lambda i: (i, 0)),
            pl.BlockSpec((TCB, 1), lambda i: (i, 0)),
            pl.BlockSpec((N_RADIAL, 2 * N_FEAT), lambda i: (0, 0)),
            pl.BlockSpec((1, 2 * N_FEAT), lambda i: (0, 0)),
        ],
        out_specs=pl.BlockSpec((TCB, 2 * N_FEAT), lambda i: (i, 0)),
        out_shape=jax.ShapeDtypeStruct((E_PAD, 2 * N_FEAT), jnp.float32),
    )(radial_p, cutoff_p, W, b)


def _splat16(val_i32):
    return jnp.full((16,), val_i32, jnp.int32)


def _sc_body(x_hbm, wf_hbm, dir_hbm, idxi_hbm, idxj_hbm, ya_hbm, yb_hbm,
             idxi_v, idxj_v, xj_v0, xj_v1, w_v0, w_v1, dir_v0, dir_v1,
             acc_v, stage_v, zr_v, mprev_v, idxi2_v, idxj2_v,
             sem_g, sem_l, sem_st, sem_f):
    cid = lax.axis_index("c")
    wid = cid * NS + lax.axis_index("s")
    base = wid * E_PC
    end_base = base + E_PC

    @pl.when(cid == 0)
    def _():
        _sc_half(x_hbm, wf_hbm, dir_hbm, idxi_hbm, idxj_hbm, ya_hbm,
                 idxi_v, idxj_v, xj_v0, xj_v1, w_v0, w_v1, dir_v0, dir_v1,
                 acc_v, stage_v, zr_v, mprev_v, idxi2_v, idxj2_v,
                 sem_g, sem_l, sem_st, sem_f, wid, base, end_base)

    @pl.when(cid == 1)
    def _():
        _sc_half(x_hbm, wf_hbm, dir_hbm, idxi_hbm, idxj_hbm, yb_hbm,
                 idxi_v, idxj_v, xj_v0, xj_v1, w_v0, w_v1, dir_v0, dir_v1,
                 acc_v, stage_v, zr_v, mprev_v, idxi2_v, idxj2_v,
                 sem_g, sem_l, sem_st, sem_f, wid, base, end_base)


def _sc_half(x_hbm, wf_hbm, dir_hbm, idxi_hbm, idxj_hbm, y_hbm,
             idxi_v, idxj_v, xj_v0, xj_v1, w_v0, w_v1, dir_v0, dir_v1,
             acc_v, stage_v, zr_v, mprev_v, idxi2_v, idxj2_v,
             sem_g, sem_l, sem_st, sem_f, wid, base, end_base):

    def _z8(i, _):
        zr_v[pl.ds(i * 16, 16)] = jnp.zeros((16,), jnp.float32)
        return 0
    lax.fori_loop(0, (8 * ROW) // 16, _z8, 0)

    def _za(i, _):
        acc_v[pl.ds(i * 16, 16)] = jnp.zeros((16,), jnp.float32)
        return 0
    lax.fori_loop(0, ROW // 16, _za, 0)

    @pl.when(wid > 0)
    def _():
        pltpu.async_copy(idxi_hbm.at[pl.ds(base - 16, 16)], mprev_v,
                         sem_l).wait()
    m = jnp.where(wid > 0, mprev_v[pl.ds(0, 16)][15], jnp.int32(-1))

    pltpu.async_copy(stage_v.at[0], y_hbm.at[pl.ds(DUMMY * ROW, ROW)],
                     sem_st.at[0])
    pltpu.async_copy(stage_v.at[1], y_hbm.at[pl.ds(DUMMY * ROW, ROW)],
                     sem_st.at[1])

    pltpu.async_copy(idxi_hbm.at[pl.ds(base, E_PC)], idxi_v, sem_l)
    pltpu.async_copy(idxj_hbm.at[pl.ds(base, E_PC)], idxj_v, sem_l)
    pltpu.make_async_copy(idxi_hbm.at[pl.ds(base, E_PC)], idxi_v,
                          sem_l).wait()
    pltpu.make_async_copy(idxj_hbm.at[pl.ds(base, E_PC)], idxj_v,
                          sem_l).wait()

    def zero_fill(wstart, cur):
        nfill = cur - wstart
        n8 = nfill // 8

        def _gap8(t, _):
            pltpu.async_copy(zr_v.at[pl.ds(0, 8 * ROW)],
                             y_hbm.at[pl.ds((wstart + t * 8) * ROW, 8 * ROW)],
                             sem_l).wait()
            return 0
        lax.fori_loop(0, n8, _gap8, 0)

        def _gap1(t, _):
            pltpu.async_copy(
                zr_v.at[pl.ds(0, ROW)],
                y_hbm.at[pl.ds((wstart + n8 * 8 + t) * ROW, ROW)],
                sem_l).wait()
            return 0
        lax.fori_loop(0, nfill - n8 * 8, _gap1, 0)

    def flush(cur, wstart, par):
        valid = cur != m
        zero_fill(wstart, cur)

        tgt = jnp.where(valid, cur, jnp.int32(DUMMY))

        def _emit(slot):
            pltpu.make_async_copy(stage_v.at[slot],
                                  y_hbm.at[pl.ds(DUMMY * ROW, ROW)],
                                  sem_st.at[slot]).wait()

            def _cp(i, _):
                v = acc_v[pl.ds(i * 16, 16)]
                stage_v[slot, pl.ds(i * 16, 16)] = v
                acc_v[pl.ds(i * 16, 16)] = jnp.zeros((16,), jnp.float32)
                return 0
            lax.fori_loop(0, ROW // 16, _cp, 0)
            pltpu.async_copy(stage_v.at[slot],
                             y_hbm.at[pl.ds(tgt * ROW, ROW)], sem_st.at[slot])

        @pl.when(par == 0)
        def _():
            _emit(0)

        @pl.when(par == 1)
        def _():
            _emit(1)

        wstart2 = jnp.where(valid, cur + 1, wstart)
        return wstart2, 1 - par

    def accumulate(p, xj_v, w_v, dir_v):
        d0 = plsc.load_gather(dir_v, [_splat16(p * S + 0)])
        d1 = plsc.load_gather(dir_v, [_splat16(p * S + 1)])
        d2 = plsc.load_gather(dir_v, [_splat16(p * S + 2)])
        d3 = plsc.load_gather(dir_v, [_splat16(p * S + 3)])

        cd1 = CNEG * d1
        cd2 = CNEG * d2
        cd3 = CNEG * d3
        for j in range(N_FEAT // 16):
            o = j * 16
            xj0 = xj_v[p, 0, pl.ds(o, 16)]
            xj1 = xj_v[p, 1, pl.ds(o, 16)]
            xj2 = xj_v[p, 2, pl.ds(o, 16)]
            xj3 = xj_v[p, 3, pl.ds(o, 16)]
            w0 = w_v[p, pl.ds(o, 16)]
            w1 = w_v[p, pl.ds(N_FEAT + o, 16)]
            u = d0 * w0
            s_ = w1 * xj0
            c0 = u * xj0 + w1 * (cd1 * xj1 + cd2 * xj2 + cd3 * xj3)
            c1 = u * xj1 + d1 * s_
            c2 = u * xj2 + d2 * s_
            c3 = u * xj3 + d3 * s_
            plsc.addupdate(acc_v.at[pl.ds(o, 16)], c0)
            plsc.addupdate(acc_v.at[pl.ds(N_FEAT + o, 16)], c1)
            plsc.addupdate(acc_v.at[pl.ds(2 * N_FEAT + o, 16)], c2)
            plsc.addupdate(acc_v.at[pl.ds(3 * N_FEAT + o, 16)], c3)

    def fetch_block(koff):
        pltpu.async_copy(wf_hbm.at[pl.ds(koff, G)], w_v0, sem_l)
        pltpu.async_copy(dir_hbm.at[pl.ds(koff * S, G * S)], dir_v0, sem_l)
        pltpu.async_copy(x_hbm.at[idxj2_v], xj_v0, sem_g)
        pltpu.make_async_copy(wf_hbm.at[pl.ds(koff, G)], w_v0, sem_l).wait()
        pltpu.make_async_copy(dir_hbm.at[pl.ds(koff * S, G * S)],
                              dir_v0, sem_l).wait()
        pltpu.make_async_copy(x_hbm.at[idxj2_v], xj_v0, sem_g).wait()

    def issue(k, w_r, dir_r, xj_r, sem):
        koff = base + k * G
        pltpu.async_copy(wf_hbm.at[pl.ds(koff, G)], w_r, sem)
        pltpu.async_copy(dir_hbm.at[pl.ds(koff * S, G * S)], dir_r, sem)
        pltpu.async_copy(x_hbm.at[idxj_v.at[pl.ds(k * G, G)]], xj_r, sem)

    def drain(k, w_r, dir_r, xj_r, sem):
        koff = base + k * G
        pltpu.make_async_copy(wf_hbm.at[pl.ds(koff, G)], w_r, sem).wait()
        pltpu.make_async_copy(dir_hbm.at[pl.ds(koff * S, G * S)],
                              dir_r, sem).wait()
        pltpu.make_async_copy(x_hbm.at[idxj_v.at[pl.ds(k * G, G)]],
                              xj_r, sem).wait()

    def process_block(k, st, xj_r, w_r, dir_r):
        def p1_edge(p, st):
            cur, wstart, par = st
            n = plsc.load_gather(idxi_v, [_splat16(k * G + p)])[0]
            boundary = n != cur

            def _fl(args):
                c_, w_, pa_ = args
                w2, pa2 = flush(c_, w_, pa_)
                return n, w2, pa2
            cur, wstart, par = lax.cond(
                boundary, _fl, lambda a: a, (cur, wstart, par))
            accumulate(p, xj_r, w_r, dir_r)
            return cur, wstart, par

        return lax.fori_loop(0, G, p1_edge, st)

    cur0 = m
    wstart0 = jnp.where(wid > 0, m + 1, jnp.int32(0))
    st = (cur0, wstart0, jnp.int32(0))

    issue(0, w_v0, dir_v0, xj_v0, sem_f.at[0])

    def pair(i, st):
        k0 = i * 2
        issue(k0 + 1, w_v1, dir_v1, xj_v1, sem_f.at[1])
        drain(k0, w_v0, dir_v0, xj_v0, sem_f.at[0])
        st = process_block(k0, st, xj_v0, w_v0, dir_v0)
        issue(k0 + 2, w_v0, dir_v0, xj_v0, sem_f.at[0])
        drain(k0 + 1, w_v1, dir_v1, xj_v1, sem_f.at[1])
        st = process_block(k0 + 1, st, xj_v1, w_v1, dir_v1)
        return st

    st = lax.fori_loop(0, (NBLK - 1) // 2, pair, st)
    drain(NBLK - 1, w_v0, dir_v0, xj_v0, sem_f.at[0])
    cur, wstart, par = process_block(NBLK - 1, st, xj_v0, w_v0, dir_v0)

    def p2_edge(p, st):
        done, cur, wstart, par = st
        n = plsc.load_gather(idxi2_v, [_splat16(p)])[0]
        active = done == 0
        boundary = (n != cur) & active

        def _fl(args):
            c_, w_, pa_ = args
            w2, pa2 = flush(c_, w_, pa_)
            return c_, w2, pa2
        cur, wstart, par = lax.cond(
            boundary, _fl, lambda a: a, (cur, wstart, par))
        done = jnp.where(active & (boundary | (cur == m)),
                         jnp.int32(1), done)

        @pl.when(done == 0)
        def _():
            accumulate(p, xj_v0, w_v0, dir_v0)
        return done, cur, wstart, par

    def p2_block(k, st):
        done, cur, wstart, par = st

        def _act(args):
            done, cur, wstart, par = args
            koff = k * G
            pltpu.async_copy(idxi_hbm.at[pl.ds(koff, G)], idxi2_v,
                             sem_l).wait()
            pltpu.async_copy(idxj_hbm.at[pl.ds(koff, G)], idxj2_v,
                             sem_l).wait()
            fetch_block(koff)
            return lax.fori_loop(0, G, p2_edge, (done, cur, wstart, par))

        return lax.cond(done == 0, _act, lambda a: a,
                        (done, cur, wstart, par))

    done, cur, wstart, par = lax.fori_loop(
        end_base // G, KMAX, p2_block,
        (jnp.int32(0), cur, wstart, par))

    @pl.when(wid == NW - 1)
    def _():
        zero_fill(wstart, jnp.int32(N_ATOMS))

    pltpu.make_async_copy(stage_v.at[0],
                          y_hbm.at[pl.ds(DUMMY * ROW, ROW)], sem_st.at[0]).wait()
    pltpu.make_async_copy(stage_v.at[1],
                          y_hbm.at[pl.ds(DUMMY * ROW, ROW)], sem_st.at[1]).wait()


def _sc_conv(x2, wfull, dir_p, idxi_p, idxj_p):
    mesh = plsc.VectorSubcoreMesh(core_axis_name="c", subcore_axis_name="s",
                                  num_cores=NC, num_subcores=NS)
    fn = pl.kernel(
        _sc_body,
        out_type=(jax.ShapeDtypeStruct(((N_ATOMS + 8) * ROW,), jnp.float32),
                  jax.ShapeDtypeStruct(((N_ATOMS + 8) * ROW,), jnp.float32)),
        mesh=mesh,
        compiler_params=pltpu.CompilerParams(needs_layout_passes=False,
                                             use_tc_tiling_on_sc=True),
        scratch_types=[
            pltpu.VMEM((E_PC,), jnp.int32),
            pltpu.VMEM((E_PC,), jnp.int32),
            pltpu.VMEM((G, S, N_FEAT), jnp.float32),
            pltpu.VMEM((G, S, N_FEAT), jnp.float32),
            pltpu.VMEM((G, 2 * N_FEAT), jnp.float32),
            pltpu.VMEM((G, 2 * N_FEAT), jnp.float32),
            pltpu.VMEM((G * S,), jnp.float32),
            pltpu.VMEM((G * S,), jnp.float32),
            pltpu.VMEM((ROW,), jnp.float32),
            pltpu.VMEM((2, ROW), jnp.float32),
            pltpu.VMEM((8 * ROW,), jnp.float32),
            pltpu.VMEM((16,), jnp.int32),
            pltpu.VMEM((G,), jnp.int32),
            pltpu.VMEM((G,), jnp.int32),
            pltpu.SemaphoreType.DMA,
            pltpu.SemaphoreType.DMA,
            pltpu.SemaphoreType.DMA((2,)),
            pltpu.SemaphoreType.DMA((2,)),
        ],
    )
    return fn(x2, wfull, dir_p, idxi_p, idxj_p)


def kernel(x, radial_ij, dir_ij, cutoff_ij, idx_i, idx_j, W, b):
    pad = PAD
    radial_p = jnp.concatenate(
        [radial_ij, jnp.zeros((pad, N_RADIAL), jnp.float32)])
    cutoff_p = jnp.concatenate([cutoff_ij, jnp.zeros((pad, 1), jnp.float32)])
    dir_p = jnp.concatenate([dir_ij, jnp.zeros((pad, S), jnp.float32)]).reshape(-1)
    idxi_p = jnp.concatenate(
        [idx_i.astype(jnp.int32),
         jnp.full((pad,), N_ATOMS, jnp.int32)])
    idxj_p = jnp.concatenate(
        [idx_j.astype(jnp.int32), jnp.zeros((pad,), jnp.int32)])
    b2 = b.reshape(1, 2 * N_FEAT)

    wfull = _wfull(radial_p, cutoff_p, W, b2)
    ya, yb = _sc_conv(x, wfull, dir_p, idxi_p, idxj_p)
    split = idx_i[N_EDGES // 2 - 1].astype(jnp.int32) + 1
    ya2 = ya[:N_ATOMS * ROW].reshape(N_ATOMS, S, N_FEAT)
    yb2 = yb[:N_ATOMS * ROW].reshape(N_ATOMS, S, N_FEAT)
    rows = jnp.arange(N_ATOMS, dtype=jnp.int32) < split
    return jnp.where(rows[:, None, None], ya2, yb2)
